# Initial kernel scaffold; baseline (speedup 1.0000x reference)
import jax
import jax.numpy as jnp
from jax.experimental import pallas as pl


def kernel(x, edge_index, W1_l, W1_r, W2_l, W2_r, b2):
    raise NotImplementedError("write your pallas kernel here")


# speedup vs baseline: 3.9618x; 3.9618x over previous
import jax
import jax.numpy as jnp
from jax import lax
from jax.experimental import pallas as pl
from jax.experimental.pallas import tpu as pltpu
import jax.experimental.pallas.tpu_sc as plsc

_N_NODES = 10000
_E = 160000
_NC = 2
_NS = 16
_CHUNK = 128
_W = 128
_E_PAD = -(-_E // (_NC * _NS * _CHUNK)) * (_NC * _NS * _CHUNK)
_NCH1 = _E_PAD // (_NS * _CHUNK)
_NCH2 = _E_PAD // (_NC * _NS * _CHUNK)
_N_PAD = 10240
_RPT = _N_PAD // _NS
_RB = 512

_MESH = plsc.VectorSubcoreMesh(core_axis_name="c", subcore_axis_name="s")


def _zero_rows(rows_v):
    zeros16 = jnp.zeros((16,), jnp.float32)

    @pl.loop(0, _CHUNK)
    def _(r):
        @pl.loop(0, _W // 16)
        def _(k):
            rows_v[r, pl.ds(k * 16, 16)] = zeros16


def _clear_acc(rows_v, acc_sh, base):
    @pl.loop(0, _RPT // _CHUNK)
    def _(i):
        pltpu.sync_copy(rows_v, acc_sh.at[pl.ds(base + i * _CHUNK, _CHUNK)])


def _sc_agg_l1():
    out_type = (jax.ShapeDtypeStruct((_N_PAD, _W), jnp.float32),
                jax.ShapeDtypeStruct((_N_PAD, _W), jnp.float32),
                jax.ShapeDtypeStruct((_N_PAD,), jnp.float32))
    scratch = [
        pltpu.VMEM((_NCH1, _CHUNK), jnp.int32),
        pltpu.VMEM((_NCH1, _CHUNK), jnp.int32),
        pltpu.VMEM((_CHUNK, _W), jnp.float32),
        pltpu.VMEM((_CHUNK,), jnp.float32),
        pltpu.VMEM((_RPT,), jnp.float32),
        pltpu.VMEM_SHARED((_N_PAD, _W), jnp.float32),
        pltpu.VMEM_SHARED((_N_PAD,), jnp.float32),
        pltpu.SemaphoreType.DMA,
    ]

    def body(src_hbm, dst_hbm, t0_hbm, t1_hbm, agg0_hbm, agg1_hbm, cnt_hbm,
             src_v, dst_v, rows_v, ones_v, zrow_v, acc_sh, cnt_sh, sem):
        c = lax.axis_index("c")
        s = lax.axis_index("s")
        base = s * _RPT
        zeros16 = jnp.zeros((16,), jnp.float32)
        ones16 = jnp.ones((16,), jnp.float32)

        _zero_rows(rows_v)
        _clear_acc(rows_v, acc_sh, base)

        @pl.loop(0, _RPT // 16)
        def _(i):
            zrow_v[pl.ds(i * 16, 16)] = zeros16

        @pl.loop(0, _CHUNK // 16)
        def _(i):
            ones_v[pl.ds(i * 16, 16)] = ones16

        pltpu.sync_copy(zrow_v, cnt_sh.at[pl.ds(base, _RPT)])

        pltpu.sync_copy(src_hbm.at[s], src_v)
        pltpu.sync_copy(dst_hbm.at[s], dst_v)

        plsc.subcore_barrier()

        @pl.loop(0, _NCH1)
        def _(j):
            @pl.when(c == 0)
            def _():
                pltpu.async_copy(t0_hbm.at[src_v.at[j]], rows_v, sem).wait()

            @pl.when(c == 1)
            def _():
                pltpu.async_copy(t1_hbm.at[src_v.at[j]], rows_v, sem).wait()

            pltpu.sync_copy(rows_v, acc_sh.at[dst_v.at[j]], add=True)

            @pl.when(c == 0)
            def _():
                pltpu.sync_copy(ones_v, cnt_sh.at[dst_v.at[j]], add=True)

        plsc.subcore_barrier()

        @pl.when(c == 0)
        def _():
            pltpu.sync_copy(acc_sh.at[pl.ds(base, _RPT)],
                            agg0_hbm.at[pl.ds(base, _RPT)])
            pltpu.sync_copy(cnt_sh.at[pl.ds(base, _RPT)],
                            cnt_hbm.at[pl.ds(base, _RPT)])

        @pl.when(c == 1)
        def _():
            pltpu.sync_copy(acc_sh.at[pl.ds(base, _RPT)],
                            agg1_hbm.at[pl.ds(base, _RPT)])

    return pl.kernel(body, out_type=out_type, mesh=_MESH,
                     scratch_types=scratch)


def _sc_agg_l2():
    out_type = (jax.ShapeDtypeStruct((_N_PAD, _W), jnp.float32),
                jax.ShapeDtypeStruct((_N_PAD, _W), jnp.float32))
    scratch = [
        pltpu.VMEM((_NCH2, _CHUNK), jnp.int32),
        pltpu.VMEM((_NCH2, _CHUNK), jnp.int32),
        pltpu.VMEM((_CHUNK, _W), jnp.float32),
        pltpu.VMEM_SHARED((_N_PAD, _W), jnp.float32),
        pltpu.SemaphoreType.DMA,
    ]

    def body(src_hbm, dst_hbm, t_hbm, agg0_hbm, agg1_hbm,
             src_v, dst_v, rows_v, acc_sh, sem):
        c = lax.axis_index("c")
        s = lax.axis_index("s")
        base = s * _RPT

        _zero_rows(rows_v)
        _clear_acc(rows_v, acc_sh, base)

        pltpu.sync_copy(src_hbm.at[c, s], src_v)
        pltpu.sync_copy(dst_hbm.at[c, s], dst_v)

        plsc.subcore_barrier()

        @pl.loop(0, _NCH2)
        def _(j):
            pltpu.async_copy(t_hbm.at[src_v.at[j]], rows_v, sem).wait()
            pltpu.sync_copy(rows_v, acc_sh.at[dst_v.at[j]], add=True)

        plsc.subcore_barrier()

        @pl.when(c == 0)
        def _():
            pltpu.sync_copy(acc_sh.at[pl.ds(base, _RPT)],
                            agg0_hbm.at[pl.ds(base, _RPT)])

        @pl.when(c == 1)
        def _():
            pltpu.sync_copy(acc_sh.at[pl.ds(base, _RPT)],
                            agg1_hbm.at[pl.ds(base, _RPT)])

    return pl.kernel(body, out_type=out_type, mesh=_MESH,
                     scratch_types=scratch)


def _tc_a_body(a0_ref, a1_ref, cnt_ref, x_ref, w1la_ref, w1lb_ref, w1r_ref,
               w2l_ref, w2r_ref, b2_ref, g_ref, r_ref):
    inv = 1.0 / jnp.maximum(cnt_ref[...], 1.0)
    a0 = a0_ref[...] * inv
    a1 = a1_ref[...] * inv
    h = (jnp.dot(a0, w1la_ref[...], preferred_element_type=jnp.float32)
         + jnp.dot(a1, w1lb_ref[...], preferred_element_type=jnp.float32)
         + jnp.dot(x_ref[...], w1r_ref[...], preferred_element_type=jnp.float32))
    h = jnp.maximum(h, 0.0)
    g = jnp.dot(h, w2l_ref[...], preferred_element_type=jnp.float32)
    g_ref[...] = jnp.concatenate(
        [g, jnp.zeros((g.shape[0], _W - g.shape[1]), jnp.float32)], axis=1)
    r_ref[...] = (jnp.dot(h, w2r_ref[...], preferred_element_type=jnp.float32)
                  + b2_ref[...])


def _tc_a(agg0, agg1, cnt2, x, w1la, w1lb, w1r, w2l, w2r, b2r):
    grid = (-(-_N_NODES // _RB),)
    f = pl.pallas_call(
        _tc_a_body,
        grid=grid,
        in_specs=[
            pl.BlockSpec((_RB, 128), lambda i: (i, 0)),
            pl.BlockSpec((_RB, 128), lambda i: (i, 0)),
            pl.BlockSpec((_RB, 1), lambda i: (i, 0)),
            pl.BlockSpec((_RB, 256), lambda i: (i, 0)),
            pl.BlockSpec((128, 256), lambda i: (0, 0)),
            pl.BlockSpec((128, 256), lambda i: (0, 0)),
            pl.BlockSpec((256, 256), lambda i: (0, 0)),
            pl.BlockSpec((256, 64), lambda i: (0, 0)),
            pl.BlockSpec((256, 64), lambda i: (0, 0)),
            pl.BlockSpec((1, 64), lambda i: (0, 0)),
        ],
        out_specs=[
            pl.BlockSpec((_RB, _W), lambda i: (i, 0)),
            pl.BlockSpec((_RB, 64), lambda i: (i, 0)),
        ],
        out_shape=[
            jax.ShapeDtypeStruct((_N_NODES, _W), jnp.float32),
            jax.ShapeDtypeStruct((_N_NODES, 64), jnp.float32),
        ],
    )
    return f(agg0, agg1, cnt2, x, w1la, w1lb, w1r, w2l, w2r, b2r)


def _tc_b_body(a0_ref, a1_ref, cnt_ref, r_ref, o_ref):
    inv = 1.0 / jnp.maximum(cnt_ref[...], 1.0)
    agg = (a0_ref[...] + a1_ref[...])[:, :64]
    o_ref[...] = agg * inv + r_ref[...]


def _tc_b(a20, a21, cnt2, r):
    grid = (-(-_N_NODES // _RB),)
    f = pl.pallas_call(
        _tc_b_body,
        grid=grid,
        in_specs=[
            pl.BlockSpec((_RB, _W), lambda i: (i, 0)),
            pl.BlockSpec((_RB, _W), lambda i: (i, 0)),
            pl.BlockSpec((_RB, 1), lambda i: (i, 0)),
            pl.BlockSpec((_RB, 64), lambda i: (i, 0)),
        ],
        out_specs=pl.BlockSpec((_RB, 64), lambda i: (i, 0)),
        out_shape=jax.ShapeDtypeStruct((_N_NODES, 64), jnp.float32),
    )
    return f(a20, a21, cnt2, r)


_agg_l1 = _sc_agg_l1()
_agg_l2 = _sc_agg_l2()


def kernel(x, edge_index, W1_l, W1_r, W2_l, W2_r, b2):
    src = edge_index[0]
    dst = edge_index[1]
    pad = _E_PAD - _E
    src_p = jnp.concatenate([src, jnp.zeros((pad,), jnp.int32)])
    dst_p = jnp.concatenate([dst, jnp.full((pad,), _N_NODES, jnp.int32)])
    src1 = src_p.reshape(_NS, _NCH1, _CHUNK)
    dst1 = dst_p.reshape(_NS, _NCH1, _CHUNK)
    src2 = src_p.reshape(_NC, _NS, _NCH2, _CHUNK)
    dst2 = dst_p.reshape(_NC, _NS, _NCH2, _CHUNK)
    x0 = x[:, :128]
    x1 = x[:, 128:]

    agg0, agg1, cnt = _agg_l1(src1, dst1, x0, x1)
    cnt2 = cnt.reshape(_N_PAD, 1)

    g, r = _tc_a(agg0, agg1, cnt2, x, W1_l[:128], W1_l[128:], W1_r,
                 W2_l, W2_r, b2.reshape(1, 64))

    a20, a21 = _agg_l2(src2, dst2, g)

    return _tc_b(a20, a21, cnt2, r)
